# baseline (speedup 1.0000x reference)
import functools

import jax
import jax.numpy as jnp
from jax import lax
from jax.experimental import pallas as pl
from jax.experimental.pallas import tpu as pltpu
from jax.experimental.pallas import tpu_sc as plsc

B = 16384
D = 64
NC = 2
NS = 16
L = 16
NW = NC * NS
BPW = B // NW
IDX_CHUNK = 128
NCHUNK = BPW // IDX_CHUNK


def _mf_body(uid_hbm, iid_hbm, utab_hbm, itab_hbm, ub_hbm, ib_hbm, gb_hbm,
             out_hbm, uid_v, iid_v, urows_v, irows_v, ub_v, ib_v, gb_v,
             out_v, sem):
  wid = lax.axis_index("s") * NC + lax.axis_index("c")
  row0 = wid * NCHUNK

  pltpu.sync_copy(uid_hbm.at[pl.ds(row0, NCHUNK)], uid_v)
  pltpu.sync_copy(iid_hbm.at[pl.ds(row0, NCHUNK)], iid_v)
  pltpu.sync_copy(gb_hbm, gb_v)

  copies = []
  for j in range(NCHUNK):
    sl = pl.ds(j * IDX_CHUNK, IDX_CHUNK)
    copies.append(pltpu.async_copy(
        utab_hbm.at[uid_v.at[j]], urows_v.at[sl], sem))
    copies.append(pltpu.async_copy(
        itab_hbm.at[iid_v.at[j]], irows_v.at[sl], sem))
    copies.append(pltpu.async_copy(
        ub_hbm.at[uid_v.at[j]], ub_v.at[sl], sem))
    copies.append(pltpu.async_copy(
        ib_hbm.at[iid_v.at[j]], ib_v.at[sl], sem))
  for cp in copies:
    cp.wait()

  gb = gb_v[...]
  lane = lax.iota(jnp.int32, L)

  def chunk_body(c, carry):
    rows = lane + c * L
    acc = gb + ub_v[pl.ds(c * L, L)] + ib_v[pl.ds(c * L, L)]
    for d in range(D):
      cols = jnp.full((L,), d, jnp.int32)
      u = plsc.load_gather(urows_v, [rows, cols])
      v = plsc.load_gather(irows_v, [rows, cols])
      acc = acc + u * v
    out_v[pl.ds(c * L, L)] = acc
    return carry

  lax.fori_loop(0, BPW // L, chunk_body, 0)
  pltpu.sync_copy(out_v, out_hbm.at[pl.ds(wid * BPW, BPW)])


@jax.jit
def _mf(uid2d, iid2d, utab, itab, ub1d, ib1d, gb16):
  mesh = plsc.VectorSubcoreMesh(core_axis_name="c", subcore_axis_name="s")
  return pl.kernel(
      _mf_body,
      out_type=jax.ShapeDtypeStruct((B,), jnp.float32),
      mesh=mesh,
      scratch_types=[
          pltpu.VMEM((NCHUNK, IDX_CHUNK), jnp.int32),
          pltpu.VMEM((NCHUNK, IDX_CHUNK), jnp.int32),
          pltpu.VMEM((BPW, D), jnp.float32),
          pltpu.VMEM((BPW, D), jnp.float32),
          pltpu.VMEM((BPW,), jnp.float32),
          pltpu.VMEM((BPW,), jnp.float32),
          pltpu.VMEM((L,), jnp.float32),
          pltpu.VMEM((BPW,), jnp.float32),
          pltpu.SemaphoreType.DMA,
      ],
      compiler_params=pltpu.CompilerParams(
          needs_layout_passes=False, use_tc_tiling_on_sc=False),
  )(uid2d, iid2d, utab, itab, ub1d, ib1d, gb16)


def kernel(user_ids, item_ids, user_emb_table, item_emb_table,
           user_bias_table, item_bias_table, global_bias):
  uid2d = user_ids.astype(jnp.int32).reshape(B // IDX_CHUNK, IDX_CHUNK)
  iid2d = item_ids.astype(jnp.int32).reshape(B // IDX_CHUNK, IDX_CHUNK)
  ub1d = user_bias_table.reshape(-1)
  ib1d = item_bias_table.reshape(-1)
  gb16 = jnp.broadcast_to(global_bias.astype(jnp.float32), (L,))
  return _mf(uid2d, iid2d, user_emb_table, item_emb_table, ub1d, ib1d, gb16)
